# Initial kernel scaffold; baseline (speedup 1.0000x reference)
import jax
import jax.numpy as jnp
from jax.experimental import pallas as pl


def kernel(patches):
    raise NotImplementedError("write your pallas kernel here")


# speedup vs baseline: 66.5545x; 66.5545x over previous
import functools

import jax
import jax.numpy as jnp
import numpy as np
from jax import lax
from jax.experimental import pallas as pl
from jax.experimental.pallas import tpu as pltpu
from jax.experimental.pallas import tpu_sc as plsc

_T, _B, _D = 576, 64, 768
_REMAIN = 144
_NROWS = _REMAIN * _B
_NW = 32
_ROWS_PER_W = _NROWS // _NW
_CHUNK = 72
_NCHUNK = _ROWS_PER_W // _CHUNK


def _constant_indexes():
    def build():
        keys = jax.random.split(jax.random.key(42), _B)
        perms = [jax.random.permutation(k, _T) for k in keys]
        fwd = jnp.stack(perms, axis=-1).astype(jnp.int32)
        bwd = jnp.argsort(fwd, axis=0).astype(jnp.int32)
        return fwd, bwd

    fwd, bwd = jax.jit(build)()
    return np.asarray(fwd), np.asarray(bwd)


_FWD, _BWD = _constant_indexes()
_FLAT_IDX = (
    (_FWD[:_REMAIN] * _B + np.arange(_B, dtype=np.int32)[None, :])
    .astype(np.int32)
    .reshape(_NW * _NCHUNK, _CHUNK)
)


@functools.cache
def _build_gather():
    @functools.partial(
        pl.kernel,
        out_type=jax.ShapeDtypeStruct((_NROWS, _D), jnp.float32),
        mesh=plsc.VectorSubcoreMesh(core_axis_name="c", subcore_axis_name="s"),
        scratch_types=[
            pltpu.VMEM((_NCHUNK, _CHUNK), jnp.int32),
            pltpu.VMEM((_CHUNK, _D), jnp.float32),
            pltpu.VMEM((_CHUNK, _D), jnp.float32),
            pltpu.SemaphoreType.DMA,
            pltpu.SemaphoreType.DMA,
        ],
    )
    def _gather_rows(src_hbm, idx_hbm, out_hbm, idx_v, buf0, buf1, sem0, sem1):
        wid = lax.axis_index("s") * 2 + lax.axis_index("c")
        base = wid * _ROWS_PER_W
        pltpu.sync_copy(idx_hbm.at[pl.ds(wid * _NCHUNK, _NCHUNK)], idx_v)
        bufs = (buf0, buf1)
        sems = (sem0, sem1)
        copies = [None] * _NCHUNK
        copies[0] = pltpu.async_copy(src_hbm.at[idx_v.at[0]], buf0, sem0)
        for j in range(_NCHUNK):
            if j + 1 < _NCHUNK:
                copies[j + 1] = pltpu.async_copy(
                    src_hbm.at[idx_v.at[j + 1]], bufs[(j + 1) % 2], sems[(j + 1) % 2]
                )
            copies[j].wait()
            pltpu.sync_copy(
                bufs[j % 2], out_hbm.at[pl.ds(base + j * _CHUNK, _CHUNK)]
            )

    return _gather_rows


def kernel(patches):
    src = patches.reshape(_T * _B, _D)
    out = _build_gather()(src, jnp.asarray(_FLAT_IDX))
    return (
        out.reshape(_REMAIN, _B, _D),
        jnp.asarray(_FWD),
        jnp.asarray(_BWD),
    )


# speedup vs baseline: 67.0911x; 1.0081x over previous
import functools

import jax
import jax.numpy as jnp
import numpy as np
from jax import lax
from jax.experimental import pallas as pl
from jax.experimental.pallas import tpu as pltpu
from jax.experimental.pallas import tpu_sc as plsc

_T, _B, _D = 576, 64, 768
_REMAIN = 144
_NROWS = _REMAIN * _B
_NW = 32
_ROWS_PER_W = _NROWS // _NW
_CHUNK = 48
_NCHUNK = _ROWS_PER_W // _CHUNK
_NBUF = 3


def _constant_indexes():
    def build():
        keys = jax.random.split(jax.random.key(42), _B)
        perms = [jax.random.permutation(k, _T) for k in keys]
        fwd = jnp.stack(perms, axis=-1).astype(jnp.int32)
        bwd = jnp.argsort(fwd, axis=0).astype(jnp.int32)
        return fwd, bwd

    fwd, bwd = jax.jit(build)()
    return np.asarray(fwd), np.asarray(bwd)


_FWD, _BWD = _constant_indexes()
_FLAT_IDX = (
    (_FWD[:_REMAIN] * _B + np.arange(_B, dtype=np.int32)[None, :])
    .astype(np.int32)
    .reshape(_NW, _NCHUNK, _CHUNK)
)


@functools.cache
def _build_gather():
    @functools.partial(
        pl.kernel,
        out_type=jax.ShapeDtypeStruct((_NROWS, _D), jnp.float32),
        mesh=plsc.VectorSubcoreMesh(core_axis_name="c", subcore_axis_name="s"),
        scratch_types=[
            pltpu.VMEM((_NCHUNK, _CHUNK), jnp.int32),
            pltpu.VMEM((_CHUNK, _D), jnp.float32),
            pltpu.VMEM((_CHUNK, _D), jnp.float32),
            pltpu.VMEM((_CHUNK, _D), jnp.float32),
            pltpu.SemaphoreType.DMA,
            pltpu.SemaphoreType.DMA,
            pltpu.SemaphoreType.DMA,
            pltpu.SemaphoreType.DMA,
            pltpu.SemaphoreType.DMA,
            pltpu.SemaphoreType.DMA,
        ],
    )
    def _gather_rows(
        src_hbm, idx_hbm, out_hbm, idx_v, b0, b1, b2, gs0, gs1, gs2, ws0, ws1, ws2
    ):
        wid = lax.axis_index("s") * 2 + lax.axis_index("c")
        base = wid * _ROWS_PER_W
        pltpu.sync_copy(idx_hbm.at[wid], idx_v)
        bufs = (b0, b1, b2)
        gsems = (gs0, gs1, gs2)
        wsems = (ws0, ws1, ws2)
        g = [None] * _NCHUNK
        w = [None] * _NCHUNK
        for j in range(_NBUF - 1):
            g[j] = pltpu.async_copy(src_hbm.at[idx_v.at[j]], bufs[j], gsems[j])
        for j in range(_NCHUNK):
            g[j].wait()
            w[j] = pltpu.async_copy(
                bufs[j % _NBUF],
                out_hbm.at[pl.ds(base + j * _CHUNK, _CHUNK)],
                wsems[j % _NBUF],
            )
            nxt = j + _NBUF - 1
            if nxt < _NCHUNK:
                if nxt - _NBUF >= 0:
                    w[nxt - _NBUF].wait()
                g[nxt] = pltpu.async_copy(
                    src_hbm.at[idx_v.at[nxt]], bufs[nxt % _NBUF], gsems[nxt % _NBUF]
                )
        for j in range(_NCHUNK - _NBUF, _NCHUNK):
            w[j].wait()

    return _gather_rows


def kernel(patches):
    src = patches.reshape(_T * _B, _D)
    out = _build_gather()(src, jnp.asarray(_FLAT_IDX))
    return (
        out.reshape(_REMAIN, _B, _D),
        jnp.asarray(_FWD),
        jnp.asarray(_BWD),
    )
